# Initial kernel scaffold; baseline (speedup 1.0000x reference)
import jax
import jax.numpy as jnp
from jax.experimental import pallas as pl


def kernel(x, mu0):
    raise NotImplementedError("write your pallas kernel here")


# speedup vs baseline: 5.5333x; 5.5333x over previous
import functools

import jax
import jax.numpy as jnp
import numpy as np
from jax.experimental import pallas as pl
from jax.experimental.pallas import tpu as pltpu

_ALPHA = 0.99
_EPS = 1e-12
_VAR0 = 40.0 ** 2
_K = 256


def _body(x_ref, mu0_ref, L_ref, a_ref, y_ref, mu_c, var_c):
    k = pl.program_id(1)

    @pl.when(k == 0)
    def _():
        mu_c[...] = mu0_ref[0]
        var_c[...] = jnp.full_like(var_c, _VAR0)

    X = x_ref[0]
    L = L_ref[...]
    a = a_ref[...]

    mu = a * mu_c[...] + jnp.dot(
        L, X, preferred_element_type=jnp.float32,
        precision=jax.lax.Precision.HIGHEST)
    D = jnp.square(X - mu)
    var = a * var_c[...] + jnp.dot(
        L, D, preferred_element_type=jnp.float32,
        precision=jax.lax.Precision.HIGHEST)
    y_ref[0] = (X - mu) / (jnp.sqrt(var) + _EPS)

    mu_c[...] = mu[-1:, :]
    var_c[...] = var[-1:, :]


@functools.lru_cache(maxsize=None)
def _coeffs():
    idx = np.arange(_K)
    diff = idx[:, None] - idx[None, :]
    L = np.where(diff >= 0, (1.0 - _ALPHA) * _ALPHA ** diff, 0.0)
    a = _ALPHA ** (idx + 1.0)
    return L.astype(np.float32), a[:, None].astype(np.float32)


def kernel(x, mu0):
    B, T, F = x.shape
    K = _K
    nk = T // K
    L_np, a_np = _coeffs()
    L = jnp.asarray(L_np)
    a = jnp.asarray(a_np)
    mu0_3d = mu0.reshape(B, 1, F)

    return pl.pallas_call(
        _body,
        out_shape=jax.ShapeDtypeStruct((B, T, F), x.dtype),
        grid=(B, nk),
        in_specs=[
            pl.BlockSpec((1, K, F), lambda b, k: (b, k, 0)),
            pl.BlockSpec((1, 1, F), lambda b, k: (b, 0, 0)),
            pl.BlockSpec((K, K), lambda b, k: (0, 0)),
            pl.BlockSpec((K, 1), lambda b, k: (0, 0)),
        ],
        out_specs=pl.BlockSpec((1, K, F), lambda b, k: (b, k, 0)),
        scratch_shapes=[
            pltpu.VMEM((1, F), jnp.float32),
            pltpu.VMEM((1, F), jnp.float32),
        ],
        compiler_params=pltpu.CompilerParams(
            dimension_semantics=("parallel", "arbitrary"),
        ),
        name="magnorm_ema",
    )(x, mu0_3d, L, a)


# speedup vs baseline: 7.0020x; 1.2654x over previous
import functools

import jax
import jax.numpy as jnp
import numpy as np
from jax.experimental import pallas as pl
from jax.experimental.pallas import tpu as pltpu

_ALPHA = 0.99
_EPS = 1e-12
_VAR0 = 40.0 ** 2
_K = 256


def _body(x_ref, mu0_ref, L_ref, a_ref, y_ref, mu_c, var_c):
    k = pl.program_id(1)

    @pl.when(k == 0)
    def _():
        mu_c[...] = mu0_ref[0]
        var_c[...] = jnp.full_like(var_c, _VAR0)

    X = x_ref[0]
    L = L_ref[...]
    a = a_ref[...]

    mu = a * mu_c[...] + jnp.dot(
        L, X, preferred_element_type=jnp.float32,
        precision=jax.lax.Precision.DEFAULT)
    D = jnp.square(X - mu)
    var = a * var_c[...] + jnp.dot(
        L, D, preferred_element_type=jnp.float32,
        precision=jax.lax.Precision.DEFAULT)
    y_ref[0] = (X - mu) / (jnp.sqrt(var) + _EPS)

    mu_c[...] = mu[-1:, :]
    var_c[...] = var[-1:, :]


@functools.lru_cache(maxsize=None)
def _coeffs():
    idx = np.arange(_K)
    diff = idx[:, None] - idx[None, :]
    L = np.where(diff >= 0, (1.0 - _ALPHA) * _ALPHA ** diff, 0.0)
    a = _ALPHA ** (idx + 1.0)
    return L.astype(np.float32), a[:, None].astype(np.float32)


def kernel(x, mu0):
    B, T, F = x.shape
    K = _K
    nk = T // K
    L_np, a_np = _coeffs()
    L = jnp.asarray(L_np)
    a = jnp.asarray(a_np)
    mu0_3d = mu0.reshape(B, 1, F)

    return pl.pallas_call(
        _body,
        out_shape=jax.ShapeDtypeStruct((B, T, F), x.dtype),
        grid=(B, nk),
        in_specs=[
            pl.BlockSpec((1, K, F), lambda b, k: (b, k, 0)),
            pl.BlockSpec((1, 1, F), lambda b, k: (b, 0, 0)),
            pl.BlockSpec((K, K), lambda b, k: (0, 0)),
            pl.BlockSpec((K, 1), lambda b, k: (0, 0)),
        ],
        out_specs=pl.BlockSpec((1, K, F), lambda b, k: (b, k, 0)),
        scratch_shapes=[
            pltpu.VMEM((1, F), jnp.float32),
            pltpu.VMEM((1, F), jnp.float32),
        ],
        compiler_params=pltpu.CompilerParams(
            dimension_semantics=("parallel", "arbitrary"),
        ),
        name="magnorm_ema",
    )(x, mu0_3d, L, a)


# speedup vs baseline: 12.6849x; 1.8116x over previous
import functools

import jax
import jax.numpy as jnp
import numpy as np
from jax.experimental import pallas as pl
from jax.experimental.pallas import tpu as pltpu

_ALPHA = 0.99
_EPS = 1e-12
_VAR0 = 40.0 ** 2
_K = 256
_FB = 13


def _body(x_ref, mu0_ref, U_ref, a_ref, y_ref, mu_c, var_c):
    k = pl.program_id(1)
    FB, B, K = x_ref.shape

    @pl.when(k == 0)
    def _():
        mu_c[...] = mu0_ref[0]
        var_c[...] = jnp.full_like(var_c, _VAR0)

    X = x_ref[...].reshape(FB * B, K)
    U = U_ref[...]
    a = a_ref[...]

    mu = a * mu_c[...] + jnp.dot(X, U, preferred_element_type=jnp.float32)
    D = jnp.square(X - mu)
    var = a * var_c[...] + jnp.dot(D, U, preferred_element_type=jnp.float32)
    y_ref[...] = ((X - mu) / (jnp.sqrt(var) + _EPS)).reshape(FB, B, K)

    mu_c[...] = mu[:, -1:]
    var_c[...] = var[:, -1:]


@functools.lru_cache(maxsize=None)
def _coeffs():
    idx = np.arange(_K)
    diff = idx[None, :] - idx[:, None]
    U = np.where(diff >= 0, (1.0 - _ALPHA) * _ALPHA ** diff, 0.0)
    a = _ALPHA ** (idx + 1.0)
    return U.astype(np.float32), a[None, :].astype(np.float32)


def kernel(x, mu0):
    B, T, F = x.shape
    K, FB = _K, _FB
    nf, nk = F // FB, T // K
    U_np, a_np = _coeffs()
    U = jnp.asarray(U_np)
    a = jnp.asarray(a_np)

    xt = jnp.transpose(x, (2, 0, 1))
    mu0_t = jnp.transpose(mu0, (1, 0)).reshape(nf, FB * B, 1)

    yt = pl.pallas_call(
        _body,
        out_shape=jax.ShapeDtypeStruct((F, B, T), x.dtype),
        grid=(nf, nk),
        in_specs=[
            pl.BlockSpec((FB, B, K), lambda f, k: (f, 0, k)),
            pl.BlockSpec((1, FB * B, 1), lambda f, k: (f, 0, 0)),
            pl.BlockSpec((K, K), lambda f, k: (0, 0)),
            pl.BlockSpec((1, K), lambda f, k: (0, 0)),
        ],
        out_specs=pl.BlockSpec((FB, B, K), lambda f, k: (f, 0, k)),
        scratch_shapes=[
            pltpu.VMEM((FB * B, 1), jnp.float32),
            pltpu.VMEM((FB * B, 1), jnp.float32),
        ],
        compiler_params=pltpu.CompilerParams(
            dimension_semantics=("parallel", "arbitrary"),
        ),
        name="magnorm_ema",
    )(xt, mu0_t, U, a)
    return jnp.transpose(yt, (1, 2, 0))


# speedup vs baseline: 21.8345x; 1.7213x over previous
import functools

import jax
import jax.numpy as jnp
import numpy as np
from jax.experimental import pallas as pl
from jax.experimental.pallas import tpu as pltpu

_ALPHA = 0.99
_EPS = 1e-12
_VAR0 = 40.0 ** 2
_K = 256
_FB = 37


def _body(x_ref, mu0_ref, U_ref, a_ref, y_ref, mu_c, var_c):
    k = pl.program_id(1)
    FB, B, K = x_ref.shape

    @pl.when(k == 0)
    def _():
        mu_c[...] = mu0_ref[0]
        var_c[...] = jnp.full_like(var_c, _VAR0)

    X = x_ref[...].reshape(FB * B, K)
    U = U_ref[...]
    a = a_ref[...]

    mu = a * mu_c[...] + jnp.dot(X, U, preferred_element_type=jnp.float32)
    D = jnp.square(X - mu)
    var = a * var_c[...] + jnp.dot(D, U, preferred_element_type=jnp.float32)
    y_ref[...] = ((X - mu) / (jnp.sqrt(var) + _EPS)).reshape(FB, B, K)

    mu_c[...] = mu[:, -1:]
    var_c[...] = var[:, -1:]


@functools.lru_cache(maxsize=None)
def _coeffs():
    idx = np.arange(_K)
    diff = idx[None, :] - idx[:, None]
    U = np.where(diff >= 0, (1.0 - _ALPHA) * _ALPHA ** diff, 0.0)
    a = _ALPHA ** (idx + 1.0)
    return U.astype(np.float32), a[None, :].astype(np.float32)


def kernel(x, mu0):
    B, T, F = x.shape
    K, FB = _K, _FB
    nf, nk = F // FB, T // K
    U_np, a_np = _coeffs()
    U = jnp.asarray(U_np)
    a = jnp.asarray(a_np)

    xt = jnp.transpose(x, (2, 0, 1))
    mu0_t = jnp.transpose(mu0, (1, 0)).reshape(nf, FB * B, 1)

    yt = pl.pallas_call(
        _body,
        out_shape=jax.ShapeDtypeStruct((F, B, T), x.dtype),
        grid=(nf, nk),
        in_specs=[
            pl.BlockSpec((FB, B, K), lambda f, k: (f, 0, k)),
            pl.BlockSpec((1, FB * B, 1), lambda f, k: (f, 0, 0)),
            pl.BlockSpec((K, K), lambda f, k: (0, 0)),
            pl.BlockSpec((1, K), lambda f, k: (0, 0)),
        ],
        out_specs=pl.BlockSpec((FB, B, K), lambda f, k: (f, 0, k)),
        scratch_shapes=[
            pltpu.VMEM((FB * B, 1), jnp.float32),
            pltpu.VMEM((FB * B, 1), jnp.float32),
        ],
        compiler_params=pltpu.CompilerParams(
            dimension_semantics=("parallel", "arbitrary"),
        ),
        name="magnorm_ema",
    )(xt, mu0_t, U, a)
    return jnp.transpose(yt, (1, 2, 0))


# speedup vs baseline: 30.7228x; 1.4071x over previous
import functools

import jax
import jax.numpy as jnp
import numpy as np
from jax.experimental import pallas as pl
from jax.experimental.pallas import tpu as pltpu

_ALPHA = 0.99
_EPS = 1e-12
_VAR0 = 40.0 ** 2
_K = 256
_KO = 1024
_FB = 37


def _body(x_ref, mu0_ref, U_ref, a_ref, y_ref, mu_c, var_c):
    k = pl.program_id(1)
    FB, B, KO = x_ref.shape
    R = FB * B

    @pl.when(k == 0)
    def _():
        mu_c[...] = mu0_ref[0]
        var_c[...] = jnp.full_like(var_c, _VAR0)

    U = U_ref[...]
    a = a_ref[...]

    for j in range(KO // _K):
        sl = pl.ds(j * _K, _K)
        X = x_ref[:, :, sl].reshape(R, _K)
        mu = a * mu_c[...] + jnp.dot(X, U, preferred_element_type=jnp.float32)
        Xc = X - mu
        var = a * var_c[...] + jnp.dot(
            jnp.square(Xc), U, preferred_element_type=jnp.float32)
        y_ref[:, :, sl] = (Xc / (jnp.sqrt(var) + _EPS)).reshape(FB, B, _K)
        mu_c[...] = mu[:, -1:]
        var_c[...] = var[:, -1:]


@functools.lru_cache(maxsize=None)
def _coeffs():
    idx = np.arange(_K)
    diff = idx[None, :] - idx[:, None]
    U = np.where(diff >= 0, (1.0 - _ALPHA) * _ALPHA ** diff, 0.0)
    a = _ALPHA ** (idx + 1.0)
    return U.astype(np.float32), a[None, :].astype(np.float32)


def kernel(x, mu0):
    B, T, F = x.shape
    K, KO, FB = _K, _KO, _FB
    nf, nk = F // FB, T // KO
    U_np, a_np = _coeffs()
    U = jnp.asarray(U_np)
    a = jnp.asarray(a_np)

    xt = jnp.transpose(x, (2, 0, 1))
    mu0_t = jnp.transpose(mu0, (1, 0)).reshape(nf, FB * B, 1)

    yt = pl.pallas_call(
        _body,
        out_shape=jax.ShapeDtypeStruct((F, B, T), x.dtype),
        grid=(nf, nk),
        in_specs=[
            pl.BlockSpec((FB, B, KO), lambda f, k: (f, 0, k)),
            pl.BlockSpec((1, FB * B, 1), lambda f, k: (f, 0, 0)),
            pl.BlockSpec((K, K), lambda f, k: (0, 0)),
            pl.BlockSpec((1, K), lambda f, k: (0, 0)),
        ],
        out_specs=pl.BlockSpec((FB, B, KO), lambda f, k: (f, 0, k)),
        scratch_shapes=[
            pltpu.VMEM((FB * B, 1), jnp.float32),
            pltpu.VMEM((FB * B, 1), jnp.float32),
        ],
        compiler_params=pltpu.CompilerParams(
            dimension_semantics=("parallel", "arbitrary"),
            vmem_limit_bytes=48 * 1024 * 1024,
        ),
        name="magnorm_ema",
    )(xt, mu0_t, U, a)
    return jnp.transpose(yt, (1, 2, 0))


# speedup vs baseline: 32.5666x; 1.0600x over previous
import functools

import jax
import jax.numpy as jnp
import numpy as np
from jax.experimental import pallas as pl
from jax.experimental.pallas import tpu as pltpu

_ALPHA = 0.99
_EPS = 1e-12
_VAR0 = 40.0 ** 2
_K = 256
_KO = 2048
_FB = 37


def _body(x_ref, mu0_ref, U_ref, a_ref, y_ref, mu_c, var_c):
    k = pl.program_id(1)
    FB, B, KO = x_ref.shape
    R = FB * B

    @pl.when(k == 0)
    def _():
        mu_c[...] = mu0_ref[0]
        var_c[...] = jnp.full_like(var_c, _VAR0)

    U = U_ref[...]
    a = a_ref[...]

    for j in range(KO // _K):
        sl = pl.ds(j * _K, _K)
        X = x_ref[:, :, sl].reshape(R, _K)
        mu = a * mu_c[...] + jnp.dot(X, U, preferred_element_type=jnp.float32)
        Xc = X - mu
        var = a * var_c[...] + jnp.dot(
            jnp.square(Xc), U, preferred_element_type=jnp.float32)
        y_ref[:, :, sl] = (Xc / (jnp.sqrt(var) + _EPS)).reshape(FB, B, _K)
        mu_c[...] = mu[:, -1:]
        var_c[...] = var[:, -1:]


@functools.lru_cache(maxsize=None)
def _coeffs():
    idx = np.arange(_K)
    diff = idx[None, :] - idx[:, None]
    U = np.where(diff >= 0, (1.0 - _ALPHA) * _ALPHA ** diff, 0.0)
    a = _ALPHA ** (idx + 1.0)
    return U.astype(np.float32), a[None, :].astype(np.float32)


def kernel(x, mu0):
    B, T, F = x.shape
    K, KO, FB = _K, _KO, _FB
    nf, nk = F // FB, T // KO
    U_np, a_np = _coeffs()
    U = jnp.asarray(U_np)
    a = jnp.asarray(a_np)

    xt = jnp.transpose(x, (2, 0, 1))
    mu0_t = jnp.transpose(mu0, (1, 0)).reshape(nf, FB * B, 1)

    yt = pl.pallas_call(
        _body,
        out_shape=jax.ShapeDtypeStruct((F, B, T), x.dtype),
        grid=(nf, nk),
        in_specs=[
            pl.BlockSpec((FB, B, KO), lambda f, k: (f, 0, k)),
            pl.BlockSpec((1, FB * B, 1), lambda f, k: (f, 0, 0)),
            pl.BlockSpec((K, K), lambda f, k: (0, 0)),
            pl.BlockSpec((1, K), lambda f, k: (0, 0)),
        ],
        out_specs=pl.BlockSpec((FB, B, KO), lambda f, k: (f, 0, k)),
        scratch_shapes=[
            pltpu.VMEM((FB * B, 1), jnp.float32),
            pltpu.VMEM((FB * B, 1), jnp.float32),
        ],
        compiler_params=pltpu.CompilerParams(
            dimension_semantics=("parallel", "arbitrary"),
            vmem_limit_bytes=48 * 1024 * 1024,
        ),
        name="magnorm_ema",
    )(xt, mu0_t, U, a)
    return jnp.transpose(yt, (1, 2, 0))
